# baseline (speedup 1.0000x reference)
import functools

import jax
import jax.numpy as jnp
from jax import lax
from jax.experimental import pallas as pl
from jax.experimental.pallas import tpu as pltpu
from jax.experimental.pallas import tpu_sc as plsc

_C = 10
_IGN = _C - 1
_B, _H, _W = 16, 512, 512
_BH = 128

_N = _B * _H * _W
_NW = 32
_PER_W = _N // _NW
_CHUNK = 32768
_NCHUNK = _PER_W // _CHUNK
_L = 16


def _tc_body(pred_ref, tgt_ref, nll_ref, cnt_ref):
    b = pl.program_id(0)
    h = pl.program_id(1)

    @pl.when((b == 0) & (h == 0))
    def _():
        nll_ref[0, 0] = 0.0
        cnt_ref[0, 0] = 0.0

    t = tgt_ref[0]
    s = jnp.zeros(t.shape, jnp.float32)
    sel = jnp.zeros(t.shape, jnp.float32)
    for c in range(_C):
        x = pred_ref[0, c]
        s = s + jnp.exp(x)
        sel = jnp.where(t == c, x, sel)
    maskf = (t != _IGN).astype(jnp.float32)
    nll = (jnp.log(s) - sel) * maskf
    nll_ref[0, 0] += jnp.sum(nll)
    cnt_ref[0, 0] += jnp.sum(maskf)


def _tc_call(pred, target):
    grid = (_B, _H // _BH)
    return pl.pallas_call(
        _tc_body,
        grid=grid,
        in_specs=[
            pl.BlockSpec((1, _C, _BH, _W), lambda b, h: (b, 0, h, 0)),
            pl.BlockSpec((1, _BH, _W), lambda b, h: (b, h, 0)),
        ],
        out_specs=[
            pl.BlockSpec((1, 1), lambda b, h: (0, 0), memory_space=pltpu.SMEM),
            pl.BlockSpec((1, 1), lambda b, h: (0, 0), memory_space=pltpu.SMEM),
        ],
        out_shape=[
            jax.ShapeDtypeStruct((1, 1), jnp.float32),
            jax.ShapeDtypeStruct((1, 1), jnp.float32),
        ],
    )(pred, target)


@functools.partial(
    pl.kernel,
    out_type=jax.ShapeDtypeStruct((_NW, _L), jnp.int32),
    mesh=plsc.VectorSubcoreMesh(core_axis_name="c", subcore_axis_name="s"),
    scratch_types=[
        pltpu.VMEM((_CHUNK,), jnp.int32),
        pltpu.VMEM((_CHUNK,), jnp.int32),
        pltpu.VMEM((_L,), jnp.int32),
        pltpu.SemaphoreType.DMA,
        pltpu.SemaphoreType.DMA,
    ],
)
def _sc_presence(tgt_hbm, out_hbm, buf0, buf1, accv, sem0, sem1):
    wid = lax.axis_index("s") * 2 + lax.axis_index("c")
    base = wid * _PER_W
    bufs = (buf0, buf1)
    sems = (sem0, sem1)
    copies = [None, None]
    copies[0] = pltpu.async_copy(tgt_hbm.at[pl.ds(base, _CHUNK)], buf0, sem0)
    acc = jnp.zeros((_L,), jnp.int32)
    one = jnp.ones((_L,), jnp.int32)
    for i in range(_NCHUNK):
        nxt = i + 1
        if nxt < _NCHUNK:
            copies[nxt % 2] = pltpu.async_copy(
                tgt_hbm.at[pl.ds(base + nxt * _CHUNK, _CHUNK)],
                bufs[nxt % 2], sems[nxt % 2])
        copies[i % 2].wait()
        buf = bufs[i % 2]

        def body(j, a, buf=buf):
            v = buf[pl.ds(j * _L, _L)]
            return a | (one << v)

        acc = lax.fori_loop(0, _CHUNK // _L, body, acc, unroll=8)
    accv[...] = acc
    pltpu.sync_copy(accv, out_hbm.at[wid])


def kernel(pred, target):
    nll, cnt = _tc_call(pred, target)
    pres = _sc_presence(target.reshape(_N))
    bits = (pres[:, :, None] >> jnp.arange(_C, dtype=jnp.int32)) & 1
    n_unique = jnp.sum(jnp.any(bits, axis=(0, 1)))
    loss = nll[0, 0] / cnt[0, 0]
    return jnp.where(n_unique < 2, 0.0 * loss, loss)


# speedup vs baseline: 1.1413x; 1.1413x over previous
import functools

import jax
import jax.numpy as jnp
from jax import lax
from jax.experimental import pallas as pl
from jax.experimental.pallas import tpu as pltpu
from jax.experimental.pallas import tpu_sc as plsc

_C = 10
_IGN = _C - 1
_B, _H, _W = 16, 512, 512
_BH = 128

_N = _B * _H * _W
_NW = 32
_ROWS_W = _H // 2
_CROWS = 64
_NCHUNK = _ROWS_W // _CROWS
_L = 16


def _tc_body(pred_ref, tgt_ref, nll_ref, cnt_ref):
    b = pl.program_id(0)
    h = pl.program_id(1)

    @pl.when((b == 0) & (h == 0))
    def _():
        nll_ref[0, 0] = 0.0
        cnt_ref[0, 0] = 0.0

    t = tgt_ref[0]
    s = jnp.zeros(t.shape, jnp.float32)
    sel = jnp.zeros(t.shape, jnp.float32)
    for c in range(_C):
        x = pred_ref[0, c]
        s = s + jnp.exp(x)
        sel = jnp.where(t == c, x, sel)
    maskf = (t != _IGN).astype(jnp.float32)
    nll = (jnp.log(s) - sel) * maskf
    nll_ref[0, 0] += jnp.sum(nll)
    cnt_ref[0, 0] += jnp.sum(maskf)


def _tc_call(pred, target):
    grid = (_B, _H // _BH)
    return pl.pallas_call(
        _tc_body,
        grid=grid,
        in_specs=[
            pl.BlockSpec((1, _C, _BH, _W), lambda b, h: (b, 0, h, 0)),
            pl.BlockSpec((1, _BH, _W), lambda b, h: (b, h, 0)),
        ],
        out_specs=[
            pl.BlockSpec((1, 1), lambda b, h: (0, 0), memory_space=pltpu.SMEM),
            pl.BlockSpec((1, 1), lambda b, h: (0, 0), memory_space=pltpu.SMEM),
        ],
        out_shape=[
            jax.ShapeDtypeStruct((1, 1), jnp.float32),
            jax.ShapeDtypeStruct((1, 1), jnp.float32),
        ],
    )(pred, target)


@functools.partial(
    pl.kernel,
    out_type=jax.ShapeDtypeStruct((_NW, _L), jnp.int32),
    mesh=plsc.VectorSubcoreMesh(core_axis_name="c", subcore_axis_name="s"),
    scratch_types=[
        pltpu.VMEM((_CROWS, _W), jnp.int32),
        pltpu.VMEM((_CROWS, _W), jnp.int32),
        pltpu.VMEM((_L,), jnp.int32),
        pltpu.SemaphoreType.DMA,
        pltpu.SemaphoreType.DMA,
    ],
)
def _sc_presence(tgt_hbm, out_hbm, buf0, buf1, accv, sem0, sem1):
    wid = lax.axis_index("s") * 2 + lax.axis_index("c")
    b = wid // 2
    r0 = (wid % 2) * _ROWS_W
    bufs = (buf0, buf1)
    sems = (sem0, sem1)
    copies = [None, None]
    copies[0] = pltpu.async_copy(
        tgt_hbm.at[b, pl.ds(r0, _CROWS), :], buf0, sem0)
    acc = jnp.zeros((_L,), jnp.int32)
    one = jnp.ones((_L,), jnp.int32)
    for i in range(_NCHUNK):
        nxt = i + 1
        if nxt < _NCHUNK:
            copies[nxt % 2] = pltpu.async_copy(
                tgt_hbm.at[b, pl.ds(r0 + nxt * _CROWS, _CROWS), :],
                bufs[nxt % 2], sems[nxt % 2])
        copies[i % 2].wait()
        buf = bufs[i % 2]

        def row(r, a, buf=buf):
            def col(c, a2):
                v = buf[r, pl.ds(c * _L, _L)]
                return a2 | (one << v)
            return lax.fori_loop(0, _W // _L, col, a, unroll=8)

        acc = lax.fori_loop(0, _CROWS, row, acc)
    accv[...] = acc
    pltpu.sync_copy(accv, out_hbm.at[wid])


def kernel(pred, target):
    nll, cnt = _tc_call(pred, target)
    pres = _sc_presence(target)
    bits = (pres[:, :, None] >> jnp.arange(_C, dtype=jnp.int32)) & 1
    n_unique = jnp.sum(jnp.any(bits, axis=(0, 1)))
    loss = nll[0, 0] / cnt[0, 0]
    return jnp.where(n_unique < 2, 0.0 * loss, loss)


# speedup vs baseline: 1.1547x; 1.0117x over previous
import functools

import jax
import jax.numpy as jnp
from jax import lax
from jax.experimental import pallas as pl
from jax.experimental.pallas import tpu as pltpu
from jax.experimental.pallas import tpu_sc as plsc

_C = 10
_IGN = _C - 1
_B, _H, _W = 16, 512, 512
_BH = 128

_KSC = 2
_BTC = _B - _KSC

_NW = 32
_L = 16

_ROWS_PW = _H // 2
_PCROWS = 64
_NPCHUNK = _ROWS_PW // _PCROWS

_WPB = _NW // _KSC
_ROWS_CW = _H // _WPB
_CCROWS = 8
_NCCHUNK = _ROWS_CW // _CCROWS

_LN2 = 0.6931471805599453
_LOG2_POLY = (0.043428907822058785, -0.4048671744185487, 1.5939013634971746,
              -3.492494279876412, 5.046876044973777, -2.786812953866816)


def _tc_body(pred_ref, tgt_ref, nll_ref, cnt_ref):
    b = pl.program_id(0)
    h = pl.program_id(1)

    @pl.when((b == 0) & (h == 0))
    def _():
        nll_ref[0, 0] = 0.0
        cnt_ref[0, 0] = 0.0

    t = tgt_ref[0]
    s = jnp.zeros(t.shape, jnp.float32)
    sel = jnp.zeros(t.shape, jnp.float32)
    for c in range(_C):
        x = pred_ref[0, c]
        s = s + jnp.exp(x)
        sel = jnp.where(t == c, x, sel)
    maskf = (t != _IGN).astype(jnp.float32)
    nll = (jnp.log(s) - sel) * maskf
    nll_ref[0, 0] += jnp.sum(nll)
    cnt_ref[0, 0] += jnp.sum(maskf)


def _tc_call(pred, target):
    grid = (_BTC, _H // _BH)
    return pl.pallas_call(
        _tc_body,
        grid=grid,
        in_specs=[
            pl.BlockSpec((1, _C, _BH, _W), lambda b, h: (b, 0, h, 0)),
            pl.BlockSpec((1, _BH, _W), lambda b, h: (b, h, 0)),
        ],
        out_specs=[
            pl.BlockSpec((1, 1), lambda b, h: (0, 0), memory_space=pltpu.SMEM),
            pl.BlockSpec((1, 1), lambda b, h: (0, 0), memory_space=pltpu.SMEM),
        ],
        out_shape=[
            jax.ShapeDtypeStruct((1, 1), jnp.float32),
            jax.ShapeDtypeStruct((1, 1), jnp.float32),
        ],
    )(pred, target)


@functools.partial(
    pl.kernel,
    out_type=jax.ShapeDtypeStruct((_NW, _L), jnp.int32),
    mesh=plsc.VectorSubcoreMesh(core_axis_name="c", subcore_axis_name="s"),
    scratch_types=[
        pltpu.VMEM((_PCROWS, _W), jnp.int32),
        pltpu.VMEM((_PCROWS, _W), jnp.int32),
        pltpu.VMEM((_L,), jnp.int32),
        pltpu.SemaphoreType.DMA,
        pltpu.SemaphoreType.DMA,
    ],
)
def _sc_presence(tgt_hbm, out_hbm, buf0, buf1, accv, sem0, sem1):
    wid = lax.axis_index("s") * 2 + lax.axis_index("c")
    b = jnp.minimum(wid // 2, _BTC - 1)
    r0 = (wid % 2) * _ROWS_PW
    bufs = (buf0, buf1)
    sems = (sem0, sem1)
    copies = [None, None]
    copies[0] = pltpu.async_copy(
        tgt_hbm.at[b, pl.ds(r0, _PCROWS), :], buf0, sem0)
    acc = jnp.zeros((_L,), jnp.int32)
    one = jnp.ones((_L,), jnp.int32)
    for i in range(_NPCHUNK):
        nxt = i + 1
        if nxt < _NPCHUNK:
            copies[nxt % 2] = pltpu.async_copy(
                tgt_hbm.at[b, pl.ds(r0 + nxt * _PCROWS, _PCROWS), :],
                bufs[nxt % 2], sems[nxt % 2])
        copies[i % 2].wait()
        buf = bufs[i % 2]

        def row(r, a, buf=buf):
            def col(c, a2):
                v = buf[r, pl.ds(c * _L, _L)]
                return a2 | (one << v)
            return lax.fori_loop(0, _W // _L, col, a, unroll=8)

        acc = lax.fori_loop(0, _PCROWS, row, acc)
    accv[...] = acc
    pltpu.sync_copy(accv, out_hbm.at[wid])


@functools.partial(
    pl.kernel,
    out_type=[
        jax.ShapeDtypeStruct((_NW, _L), jnp.float32),
        jax.ShapeDtypeStruct((_NW, _L), jnp.float32),
        jax.ShapeDtypeStruct((_NW, _L), jnp.int32),
    ],
    mesh=plsc.VectorSubcoreMesh(core_axis_name="c", subcore_axis_name="s"),
    scratch_types=[
        pltpu.VMEM((_C, _CCROWS, _W), jnp.float32),
        pltpu.VMEM((_C, _CCROWS, _W), jnp.float32),
        pltpu.VMEM((_CCROWS, _W), jnp.int32),
        pltpu.VMEM((_CCROWS, _W), jnp.int32),
        pltpu.VMEM((_L,), jnp.float32),
        pltpu.VMEM((_L,), jnp.float32),
        pltpu.VMEM((_L,), jnp.int32),
        pltpu.SemaphoreType.DMA,
        pltpu.SemaphoreType.DMA,
    ],
)
def _sc_ce(pred_hbm, tgt_hbm, nll_hbm, cnt_hbm, pres_hbm,
           bp0, bp1, bt0, bt1, nllv, cntv, presv, sem0, sem1):
    wid = lax.axis_index("s") * 2 + lax.axis_index("c")
    b = _BTC + wid // _WPB
    r0 = (wid % _WPB) * _ROWS_CW
    bps = (bp0, bp1)
    bts = (bt0, bt1)
    sems = (sem0, sem1)

    def fire(k, which):
        cs = []
        for c in range(_C):
            cs.append(pltpu.async_copy(
                pred_hbm.at[b, c, pl.ds(r0 + k * _CCROWS, _CCROWS), :],
                bps[which].at[c], sems[which]))
        cs.append(pltpu.async_copy(
            tgt_hbm.at[b, pl.ds(r0 + k * _CCROWS, _CCROWS), :],
            bts[which], sems[which]))
        return cs

    copies = [None, None]
    copies[0] = fire(0, 0)

    nll = jnp.zeros((_L,), jnp.float32)
    cnt = jnp.zeros((_L,), jnp.float32)
    pres = jnp.zeros((_L,), jnp.int32)
    one = jnp.ones((_L,), jnp.int32)
    zf = jnp.zeros((_L,), jnp.float32)
    onef = jnp.ones((_L,), jnp.float32)
    cvecs = [jnp.full((_L,), c, jnp.int32) for c in range(_C)]
    ign_v = jnp.full((_L,), _IGN, jnp.int32)
    c23 = jnp.full((_L,), 23, jnp.int32)
    c127 = jnp.full((_L,), 127, jnp.int32)
    cmant = jnp.full((_L,), 0x7FFFFF, jnp.int32)
    cone_f = jnp.full((_L,), 0x3F800000, jnp.int32)
    poly = [jnp.full((_L,), c, jnp.float32) for c in _LOG2_POLY]
    ln2_v = jnp.full((_L,), _LN2, jnp.float32)

    for i in range(_NCCHUNK):
        nxt = i + 1
        if nxt < _NCCHUNK:
            copies[nxt % 2] = fire(nxt, nxt % 2)
        for cp in copies[i % 2]:
            cp.wait()
        bp = bps[i % 2]
        bt = bts[i % 2]

        def row(r, carry, bp=bp, bt=bt):
            def col(cc, carry2):
                nll_a, cnt_a, pres_a = carry2
                t = bt[r, pl.ds(cc * _L, _L)]
                s = zf
                sel = zf
                for c in range(_C):
                    x = bp[c, r, pl.ds(cc * _L, _L)]
                    s = s + jnp.exp(x)
                    sel = jnp.where(t == cvecs[c], x, sel)
                bits = lax.bitcast_convert_type(s, jnp.int32)
                e = (bits >> c23) - c127
                mant = lax.bitcast_convert_type(
                    (bits & cmant) | cone_f, jnp.float32)
                p = poly[0]
                for coef in poly[1:]:
                    p = p * mant + coef
                ln_s = (p + e.astype(jnp.float32)) * ln2_v
                maskf = jnp.where(t != ign_v, onef, zf)
                nll_a = nll_a + (ln_s - sel) * maskf
                cnt_a = cnt_a + maskf
                pres_a = pres_a | (one << t)
                return (nll_a, cnt_a, pres_a)
            return lax.fori_loop(0, _W // _L, col, carry, unroll=2)

        nll, cnt, pres = lax.fori_loop(0, _CCROWS, row, (nll, cnt, pres))

    nllv[...] = nll
    cntv[...] = cnt
    presv[...] = pres
    pltpu.sync_copy(nllv, nll_hbm.at[wid])
    pltpu.sync_copy(cntv, cnt_hbm.at[wid])
    pltpu.sync_copy(presv, pres_hbm.at[wid])


def kernel(pred, target):
    nll_tc, cnt_tc = _tc_call(pred, target)
    nll_sc, cnt_sc, pres_ce = _sc_ce(pred, target)
    pres_tc = _sc_presence(target)
    nll = nll_tc[0, 0] + jnp.sum(nll_sc)
    cnt = cnt_tc[0, 0] + jnp.sum(cnt_sc)
    pres = jnp.concatenate([pres_tc.ravel(), pres_ce.ravel()])
    bits = (pres[:, None] >> jnp.arange(_C, dtype=jnp.int32)) & 1
    n_unique = jnp.sum(jnp.any(bits, axis=0))
    loss = nll / cnt
    return jnp.where(n_unique < 2, 0.0 * loss, loss)


# speedup vs baseline: 1.2204x; 1.0569x over previous
import functools

import jax
import jax.numpy as jnp
from jax import lax
from jax.experimental import pallas as pl
from jax.experimental.pallas import tpu as pltpu
from jax.experimental.pallas import tpu_sc as plsc

_C = 10
_IGN = _C - 1
_B, _H, _W = 16, 512, 512
_BH = 128

_KSC = 4
_BTC = _B - _KSC

_NW = 32
_L = 16

_ROWS_PW = _H // 2
_PCROWS = 64
_NPCHUNK = _ROWS_PW // _PCROWS

_WPB = _NW // _KSC
_ROWS_CW = _H // _WPB
_CCROWS = 8
_NCCHUNK = _ROWS_CW // _CCROWS

_LN2 = 0.6931471805599453
_LOG2_POLY = (0.043428907822058785, -0.4048671744185487, 1.5939013634971746,
              -3.492494279876412, 5.046876044973777, -2.786812953866816)


def _tc_body(pred_ref, tgt_ref, nll_ref, cnt_ref):
    b = pl.program_id(0)
    h = pl.program_id(1)

    @pl.when((b == 0) & (h == 0))
    def _():
        nll_ref[0, 0] = 0.0
        cnt_ref[0, 0] = 0.0

    t = tgt_ref[0]
    s = jnp.zeros(t.shape, jnp.float32)
    sel = jnp.zeros(t.shape, jnp.float32)
    for c in range(_C):
        x = pred_ref[0, c]
        s = s + jnp.exp(x)
        sel = jnp.where(t == c, x, sel)
    maskf = (t != _IGN).astype(jnp.float32)
    nll = (jnp.log(s) - sel) * maskf
    nll_ref[0, 0] += jnp.sum(nll)
    cnt_ref[0, 0] += jnp.sum(maskf)


def _tc_call(pred, target):
    grid = (_BTC, _H // _BH)
    return pl.pallas_call(
        _tc_body,
        grid=grid,
        in_specs=[
            pl.BlockSpec((1, _C, _BH, _W), lambda b, h: (b, 0, h, 0)),
            pl.BlockSpec((1, _BH, _W), lambda b, h: (b, h, 0)),
        ],
        out_specs=[
            pl.BlockSpec((1, 1), lambda b, h: (0, 0), memory_space=pltpu.SMEM),
            pl.BlockSpec((1, 1), lambda b, h: (0, 0), memory_space=pltpu.SMEM),
        ],
        out_shape=[
            jax.ShapeDtypeStruct((1, 1), jnp.float32),
            jax.ShapeDtypeStruct((1, 1), jnp.float32),
        ],
    )(pred, target)


@functools.partial(
    pl.kernel,
    out_type=jax.ShapeDtypeStruct((_NW, _L), jnp.int32),
    mesh=plsc.VectorSubcoreMesh(core_axis_name="c", subcore_axis_name="s"),
    scratch_types=[
        pltpu.VMEM((_PCROWS, _W), jnp.int32),
        pltpu.VMEM((_PCROWS, _W), jnp.int32),
        pltpu.VMEM((_L,), jnp.int32),
        pltpu.SemaphoreType.DMA,
        pltpu.SemaphoreType.DMA,
    ],
)
def _sc_presence(tgt_hbm, out_hbm, buf0, buf1, accv, sem0, sem1):
    wid = lax.axis_index("s") * 2 + lax.axis_index("c")
    b = jnp.minimum(wid // 2, _BTC - 1)
    r0 = (wid % 2) * _ROWS_PW
    bufs = (buf0, buf1)
    sems = (sem0, sem1)
    copies = [None, None]
    copies[0] = pltpu.async_copy(
        tgt_hbm.at[b, pl.ds(r0, _PCROWS), :], buf0, sem0)
    acc = jnp.zeros((_L,), jnp.int32)
    one = jnp.ones((_L,), jnp.int32)
    for i in range(_NPCHUNK):
        nxt = i + 1
        if nxt < _NPCHUNK:
            copies[nxt % 2] = pltpu.async_copy(
                tgt_hbm.at[b, pl.ds(r0 + nxt * _PCROWS, _PCROWS), :],
                bufs[nxt % 2], sems[nxt % 2])
        copies[i % 2].wait()
        buf = bufs[i % 2]

        def row(r, a, buf=buf):
            def col(c, a2):
                v = buf[r, pl.ds(c * _L, _L)]
                return a2 | (one << v)
            return lax.fori_loop(0, _W // _L, col, a, unroll=8)

        acc = lax.fori_loop(0, _PCROWS, row, acc)
    accv[...] = acc
    pltpu.sync_copy(accv, out_hbm.at[wid])


@functools.partial(
    pl.kernel,
    out_type=[
        jax.ShapeDtypeStruct((_NW, _L), jnp.float32),
        jax.ShapeDtypeStruct((_NW, _L), jnp.float32),
        jax.ShapeDtypeStruct((_NW, _L), jnp.int32),
    ],
    mesh=plsc.VectorSubcoreMesh(core_axis_name="c", subcore_axis_name="s"),
    scratch_types=[
        pltpu.VMEM((_C, _CCROWS, _W), jnp.float32),
        pltpu.VMEM((_C, _CCROWS, _W), jnp.float32),
        pltpu.VMEM((_CCROWS, _W), jnp.int32),
        pltpu.VMEM((_CCROWS, _W), jnp.int32),
        pltpu.VMEM((_L,), jnp.float32),
        pltpu.VMEM((_L,), jnp.float32),
        pltpu.VMEM((_L,), jnp.int32),
        pltpu.SemaphoreType.DMA,
        pltpu.SemaphoreType.DMA,
    ],
)
def _sc_ce(pred_hbm, tgt_hbm, nll_hbm, cnt_hbm, pres_hbm,
           bp0, bp1, bt0, bt1, nllv, cntv, presv, sem0, sem1):
    wid = lax.axis_index("s") * 2 + lax.axis_index("c")
    b = _BTC + wid // _WPB
    r0 = (wid % _WPB) * _ROWS_CW
    bps = (bp0, bp1)
    bts = (bt0, bt1)
    sems = (sem0, sem1)

    def fire(k, which):
        cs = []
        for c in range(_C):
            cs.append(pltpu.async_copy(
                pred_hbm.at[b, c, pl.ds(r0 + k * _CCROWS, _CCROWS), :],
                bps[which].at[c], sems[which]))
        cs.append(pltpu.async_copy(
            tgt_hbm.at[b, pl.ds(r0 + k * _CCROWS, _CCROWS), :],
            bts[which], sems[which]))
        return cs

    copies = [None, None]
    copies[0] = fire(0, 0)

    nll = jnp.zeros((_L,), jnp.float32)
    cnt = jnp.zeros((_L,), jnp.float32)
    pres = jnp.zeros((_L,), jnp.int32)
    one = jnp.ones((_L,), jnp.int32)
    zf = jnp.zeros((_L,), jnp.float32)
    onef = jnp.ones((_L,), jnp.float32)
    cvecs = [jnp.full((_L,), c, jnp.int32) for c in range(_C)]
    ign_v = jnp.full((_L,), _IGN, jnp.int32)
    c23 = jnp.full((_L,), 23, jnp.int32)
    c127 = jnp.full((_L,), 127, jnp.int32)
    cmant = jnp.full((_L,), 0x7FFFFF, jnp.int32)
    cone_f = jnp.full((_L,), 0x3F800000, jnp.int32)
    poly = [jnp.full((_L,), c, jnp.float32) for c in _LOG2_POLY]
    ln2_v = jnp.full((_L,), _LN2, jnp.float32)

    for i in range(_NCCHUNK):
        nxt = i + 1
        if nxt < _NCCHUNK:
            copies[nxt % 2] = fire(nxt, nxt % 2)
        for cp in copies[i % 2]:
            cp.wait()
        bp = bps[i % 2]
        bt = bts[i % 2]

        def row(r, carry, bp=bp, bt=bt):
            def col(cc, carry2):
                nll_a, cnt_a, pres_a = carry2
                t = bt[r, pl.ds(cc * _L, _L)]
                s = zf
                sel = zf
                for c in range(_C):
                    x = bp[c, r, pl.ds(cc * _L, _L)]
                    s = s + jnp.exp(x)
                    sel = jnp.where(t == cvecs[c], x, sel)
                bits = lax.bitcast_convert_type(s, jnp.int32)
                e = (bits >> c23) - c127
                mant = lax.bitcast_convert_type(
                    (bits & cmant) | cone_f, jnp.float32)
                p = poly[0]
                for coef in poly[1:]:
                    p = p * mant + coef
                ln_s = (p + e.astype(jnp.float32)) * ln2_v
                maskf = jnp.where(t != ign_v, onef, zf)
                nll_a = nll_a + (ln_s - sel) * maskf
                cnt_a = cnt_a + maskf
                pres_a = pres_a | (one << t)
                return (nll_a, cnt_a, pres_a)
            return lax.fori_loop(0, _W // _L, col, carry, unroll=4)

        nll, cnt, pres = lax.fori_loop(0, _CCROWS, row, (nll, cnt, pres))

    nllv[...] = nll
    cntv[...] = cnt
    presv[...] = pres
    pltpu.sync_copy(nllv, nll_hbm.at[wid])
    pltpu.sync_copy(cntv, cnt_hbm.at[wid])
    pltpu.sync_copy(presv, pres_hbm.at[wid])


def kernel(pred, target):
    nll_tc, cnt_tc = _tc_call(pred, target)
    nll_sc, cnt_sc, pres_ce = _sc_ce(pred, target)
    pres_tc = _sc_presence(target)
    nll = nll_tc[0, 0] + jnp.sum(nll_sc)
    cnt = cnt_tc[0, 0] + jnp.sum(cnt_sc)
    pres = jnp.concatenate([pres_tc.ravel(), pres_ce.ravel()])
    bits = (pres[:, None] >> jnp.arange(_C, dtype=jnp.int32)) & 1
    n_unique = jnp.sum(jnp.any(bits, axis=0))
    loss = nll / cnt
    return jnp.where(n_unique < 2, 0.0 * loss, loss)


# speedup vs baseline: 1.2231x; 1.0023x over previous
import functools

import jax
import jax.numpy as jnp
from jax import lax
from jax.experimental import pallas as pl
from jax.experimental.pallas import tpu as pltpu
from jax.experimental.pallas import tpu_sc as plsc

_C = 10
_IGN = _C - 1
_B, _H, _W = 16, 512, 512
_BH = 128

_KSC = 4
_BTC = _B - _KSC

_NW = 32
_L = 16

_ROWS_PW = _H // 2
_PCROWS = 64
_NPCHUNK = _ROWS_PW // _PCROWS

_WPB = _NW // _KSC
_ROWS_CW = _H // _WPB
_CCROWS = 8
_NCCHUNK = _ROWS_CW // _CCROWS

_LN2 = 0.6931471805599453
_LOG2_POLY = (-0.07915036575313755, 0.6288157291847285, -2.081060203458998,
              4.028372766846473, -2.4967737679054225)


def _tc_body(pred_ref, tgt_ref, nll_ref, cnt_ref):
    b = pl.program_id(0)
    h = pl.program_id(1)

    @pl.when((b == 0) & (h == 0))
    def _():
        nll_ref[0, 0] = 0.0
        cnt_ref[0, 0] = 0.0

    t = tgt_ref[0]
    s = jnp.zeros(t.shape, jnp.float32)
    sel = jnp.zeros(t.shape, jnp.float32)
    for c in range(_C):
        x = pred_ref[0, c]
        s = s + jnp.exp(x)
        sel = jnp.where(t == c, x, sel)
    maskf = (t != _IGN).astype(jnp.float32)
    nll = (jnp.log(s) - sel) * maskf
    nll_ref[0, 0] += jnp.sum(nll)
    cnt_ref[0, 0] += jnp.sum(maskf)


def _tc_call(pred, target):
    grid = (_BTC, _H // _BH)
    return pl.pallas_call(
        _tc_body,
        grid=grid,
        in_specs=[
            pl.BlockSpec((1, _C, _BH, _W), lambda b, h: (b, 0, h, 0)),
            pl.BlockSpec((1, _BH, _W), lambda b, h: (b, h, 0)),
        ],
        out_specs=[
            pl.BlockSpec((1, 1), lambda b, h: (0, 0), memory_space=pltpu.SMEM),
            pl.BlockSpec((1, 1), lambda b, h: (0, 0), memory_space=pltpu.SMEM),
        ],
        out_shape=[
            jax.ShapeDtypeStruct((1, 1), jnp.float32),
            jax.ShapeDtypeStruct((1, 1), jnp.float32),
        ],
    )(pred, target)


@functools.partial(
    pl.kernel,
    out_type=jax.ShapeDtypeStruct((_NW, _L), jnp.int32),
    mesh=plsc.VectorSubcoreMesh(core_axis_name="c", subcore_axis_name="s"),
    scratch_types=[
        pltpu.VMEM((_PCROWS, _W), jnp.int32),
        pltpu.VMEM((_PCROWS, _W), jnp.int32),
        pltpu.VMEM((_L,), jnp.int32),
        pltpu.SemaphoreType.DMA,
        pltpu.SemaphoreType.DMA,
    ],
)
def _sc_presence(tgt_hbm, out_hbm, buf0, buf1, accv, sem0, sem1):
    wid = lax.axis_index("s") * 2 + lax.axis_index("c")
    b = jnp.minimum(wid // 2, _BTC - 1)
    r0 = (wid % 2) * _ROWS_PW
    bufs = (buf0, buf1)
    sems = (sem0, sem1)
    copies = [None, None]
    copies[0] = pltpu.async_copy(
        tgt_hbm.at[b, pl.ds(r0, _PCROWS), :], buf0, sem0)
    acc = jnp.zeros((_L,), jnp.int32)
    one = jnp.ones((_L,), jnp.int32)
    for i in range(_NPCHUNK):
        nxt = i + 1
        if nxt < _NPCHUNK:
            copies[nxt % 2] = pltpu.async_copy(
                tgt_hbm.at[b, pl.ds(r0 + nxt * _PCROWS, _PCROWS), :],
                bufs[nxt % 2], sems[nxt % 2])
        copies[i % 2].wait()
        buf = bufs[i % 2]

        def row(r, a, buf=buf):
            def col(c, a2):
                v = buf[r, pl.ds(c * _L, _L)]
                return a2 | (one << v)
            return lax.fori_loop(0, _W // _L, col, a, unroll=8)

        acc = lax.fori_loop(0, _PCROWS, row, acc)
    accv[...] = acc
    pltpu.sync_copy(accv, out_hbm.at[wid])


@functools.partial(
    pl.kernel,
    out_type=[
        jax.ShapeDtypeStruct((_NW, _L), jnp.float32),
        jax.ShapeDtypeStruct((_NW, _L), jnp.float32),
        jax.ShapeDtypeStruct((_NW, _L), jnp.int32),
    ],
    mesh=plsc.VectorSubcoreMesh(core_axis_name="c", subcore_axis_name="s"),
    scratch_types=[
        pltpu.VMEM((_C, _CCROWS, _W), jnp.float32),
        pltpu.VMEM((_C, _CCROWS, _W), jnp.float32),
        pltpu.VMEM((_CCROWS, _W), jnp.int32),
        pltpu.VMEM((_CCROWS, _W), jnp.int32),
        pltpu.VMEM((_L,), jnp.float32),
        pltpu.VMEM((_L,), jnp.float32),
        pltpu.VMEM((_L,), jnp.int32),
        pltpu.SemaphoreType.DMA,
        pltpu.SemaphoreType.DMA,
    ],
)
def _sc_ce(pred_hbm, tgt_hbm, nll_hbm, cnt_hbm, pres_hbm,
           bp0, bp1, bt0, bt1, nllv, cntv, presv, sem0, sem1):
    wid = lax.axis_index("s") * 2 + lax.axis_index("c")
    b = _BTC + wid // _WPB
    r0 = (wid % _WPB) * _ROWS_CW
    bps = (bp0, bp1)
    bts = (bt0, bt1)
    sems = (sem0, sem1)

    def fire(k, which):
        cs = [pltpu.async_copy(
            pred_hbm.at[b, :, pl.ds(r0 + k * _CCROWS, _CCROWS), :],
            bps[which], sems[which])]
        cs.append(pltpu.async_copy(
            tgt_hbm.at[b, pl.ds(r0 + k * _CCROWS, _CCROWS), :],
            bts[which], sems[which]))
        return cs

    copies = [None, None]
    copies[0] = fire(0, 0)

    nll = jnp.zeros((_L,), jnp.float32)
    cnt = jnp.zeros((_L,), jnp.float32)
    pres = jnp.zeros((_L,), jnp.int32)
    one = jnp.ones((_L,), jnp.int32)
    zf = jnp.zeros((_L,), jnp.float32)
    onef = jnp.ones((_L,), jnp.float32)
    ign_v = jnp.full((_L,), _IGN, jnp.int32)
    cvecs = [jnp.full((_L,), c, jnp.int32) for c in range(_C)]
    c23 = jnp.full((_L,), 23, jnp.int32)
    c127 = jnp.full((_L,), 127, jnp.int32)
    cmant = jnp.full((_L,), 0x7FFFFF, jnp.int32)
    cone_f = jnp.full((_L,), 0x3F800000, jnp.int32)
    poly = [jnp.full((_L,), c, jnp.float32) for c in _LOG2_POLY]
    ln2_v = jnp.full((_L,), _LN2, jnp.float32)

    for i in range(_NCCHUNK):
        nxt = i + 1
        if nxt < _NCCHUNK:
            copies[nxt % 2] = fire(nxt, nxt % 2)
        for cp in copies[i % 2]:
            cp.wait()
        bp = bps[i % 2]
        bt = bts[i % 2]

        def row(r, carry, bp=bp, bt=bt):
            def col(cc, carry2):
                nll_a, cnt_a, pres_a = carry2
                t = bt[r, pl.ds(cc * _L, _L)]
                s = zf
                for c in range(_C):
                    x = bp[c, r, pl.ds(cc * _L, _L)]
                    s = s + jnp.exp(x)
                sel = zf
                for c in range(_C):
                    x = bp[c, r, pl.ds(cc * _L, _L)]
                    sel = jnp.where(t == cvecs[c], x, sel)
                bits = lax.bitcast_convert_type(s, jnp.int32)
                e = (bits >> c23) - c127
                mant = lax.bitcast_convert_type(
                    (bits & cmant) | cone_f, jnp.float32)
                p = poly[0]
                for coef in poly[1:]:
                    p = p * mant + coef
                ln_s = (p + e.astype(jnp.float32)) * ln2_v
                maskf = jnp.where(t != ign_v, onef, zf)
                nll_a = nll_a + (ln_s - sel) * maskf
                cnt_a = cnt_a + maskf
                pres_a = pres_a | (one << t)
                return (nll_a, cnt_a, pres_a)
            return lax.fori_loop(0, _W // _L, col, carry, unroll=2)

        nll, cnt, pres = lax.fori_loop(0, _CCROWS, row, (nll, cnt, pres))

    nllv[...] = nll
    cntv[...] = cnt
    presv[...] = pres
    pltpu.sync_copy(nllv, nll_hbm.at[wid])
    pltpu.sync_copy(cntv, cnt_hbm.at[wid])
    pltpu.sync_copy(presv, pres_hbm.at[wid])


def kernel(pred, target):
    nll_tc, cnt_tc = _tc_call(pred, target)
    nll_sc, cnt_sc, pres_ce = _sc_ce(pred, target)
    pres_tc = _sc_presence(target)
    nll = nll_tc[0, 0] + jnp.sum(nll_sc)
    cnt = cnt_tc[0, 0] + jnp.sum(cnt_sc)
    pres = jnp.concatenate([pres_tc.ravel(), pres_ce.ravel()])
    bits = (pres[:, None] >> jnp.arange(_C, dtype=jnp.int32)) & 1
    n_unique = jnp.sum(jnp.any(bits, axis=0))
    loss = nll / cnt
    return jnp.where(n_unique < 2, 0.0 * loss, loss)


# speedup vs baseline: 1.3479x; 1.1020x over previous
import functools

import jax
import jax.numpy as jnp
from jax import lax
from jax.experimental import pallas as pl
from jax.experimental.pallas import tpu as pltpu
from jax.experimental.pallas import tpu_sc as plsc

_C = 10
_IGN = _C - 1
_B, _H, _W = 16, 512, 512
_BH = 128

_KSC = 4
_BTC = _B - _KSC

_NW = 32
_L = 16

_WPB = _NW // _KSC
_ROWS_CW = _H // _WPB
_CCROWS = 8
_NCCHUNK = _ROWS_CW // _CCROWS

_PROWS_W = _BTC * _H // _NW
_P2ROWS = 32
_NP2 = _PROWS_W // _P2ROWS

_LN2 = 0.6931471805599453
_LOG2_POLY = (-0.07915036575313755, 0.6288157291847285, -2.081060203458998,
              4.028372766846473, -2.4967737679054225)


def _tc_body(pred_ref, tgt_ref, nll_ref, cnt_ref):
    b = pl.program_id(0)
    h = pl.program_id(1)

    @pl.when((b == 0) & (h == 0))
    def _():
        nll_ref[0, 0] = 0.0
        cnt_ref[0, 0] = 0.0

    t = tgt_ref[0]
    s = jnp.zeros(t.shape, jnp.float32)
    sel = jnp.zeros(t.shape, jnp.float32)
    for c in range(_C):
        x = pred_ref[0, c]
        s = s + jnp.exp(x)
        sel = jnp.where(t == c, x, sel)
    maskf = (t != _IGN).astype(jnp.float32)
    nll = (jnp.log(s) - sel) * maskf
    nll_ref[0, 0] += jnp.sum(nll)
    cnt_ref[0, 0] += jnp.sum(maskf)


def _tc_call(pred, target):
    grid = (_BTC, _H // _BH)
    return pl.pallas_call(
        _tc_body,
        grid=grid,
        in_specs=[
            pl.BlockSpec((1, _C, _BH, _W), lambda b, h: (b, 0, h, 0)),
            pl.BlockSpec((1, _BH, _W), lambda b, h: (b, h, 0)),
        ],
        out_specs=[
            pl.BlockSpec((1, 1), lambda b, h: (0, 0), memory_space=pltpu.SMEM),
            pl.BlockSpec((1, 1), lambda b, h: (0, 0), memory_space=pltpu.SMEM),
        ],
        out_shape=[
            jax.ShapeDtypeStruct((1, 1), jnp.float32),
            jax.ShapeDtypeStruct((1, 1), jnp.float32),
        ],
    )(pred, target)


@functools.partial(
    pl.kernel,
    out_type=[
        jax.ShapeDtypeStruct((_NW, _L), jnp.float32),
        jax.ShapeDtypeStruct((_NW, _L), jnp.float32),
        jax.ShapeDtypeStruct((_NW, _L), jnp.int32),
    ],
    mesh=plsc.VectorSubcoreMesh(core_axis_name="c", subcore_axis_name="s"),
    scratch_types=[
        pltpu.VMEM((_C, _CCROWS, _W), jnp.float32),
        pltpu.VMEM((_C, _CCROWS, _W), jnp.float32),
        pltpu.VMEM((_CCROWS, _W), jnp.int32),
        pltpu.VMEM((_CCROWS, _W), jnp.int32),
        pltpu.VMEM((_P2ROWS, _W), jnp.int32),
        pltpu.VMEM((_P2ROWS, _W), jnp.int32),
        pltpu.VMEM((_L,), jnp.float32),
        pltpu.VMEM((_L,), jnp.float32),
        pltpu.VMEM((_L,), jnp.int32),
        pltpu.SemaphoreType.DMA,
        pltpu.SemaphoreType.DMA,
    ],
)
def _sc_main(pred_hbm, tgt_hbm, nll_hbm, cnt_hbm, pres_hbm,
             bp0, bp1, bt0, bt1, pb0, pb1, nllv, cntv, presv, sem0, sem1):
    wid = lax.axis_index("s") * 2 + lax.axis_index("c")
    b = _BTC + wid // _WPB
    r0 = (wid % _WPB) * _ROWS_CW
    bps = (bp0, bp1)
    bts = (bt0, bt1)
    pbs = (pb0, pb1)
    sems = (sem0, sem1)

    def fire(k, which):
        cs = [pltpu.async_copy(
            pred_hbm.at[b, :, pl.ds(r0 + k * _CCROWS, _CCROWS), :],
            bps[which], sems[which])]
        cs.append(pltpu.async_copy(
            tgt_hbm.at[b, pl.ds(r0 + k * _CCROWS, _CCROWS), :],
            bts[which], sems[which]))
        return cs

    pg0 = wid * _PROWS_W

    def pfire(k, which):
        g = pg0 + k * _P2ROWS
        return pltpu.async_copy(
            tgt_hbm.at[g // _H, pl.ds(g % _H, _P2ROWS), :],
            pbs[which], sems[which])

    copies = [None, None]
    copies[0] = fire(0, 0)

    nll = jnp.zeros((_L,), jnp.float32)
    cnt = jnp.zeros((_L,), jnp.float32)
    pres = jnp.zeros((_L,), jnp.int32)
    one = jnp.ones((_L,), jnp.int32)
    zf = jnp.zeros((_L,), jnp.float32)
    onef = jnp.ones((_L,), jnp.float32)
    ign_v = jnp.full((_L,), _IGN, jnp.int32)
    cvecs = [jnp.full((_L,), c, jnp.int32) for c in range(_C)]
    c23 = jnp.full((_L,), 23, jnp.int32)
    c127 = jnp.full((_L,), 127, jnp.int32)
    cmant = jnp.full((_L,), 0x7FFFFF, jnp.int32)
    cone_f = jnp.full((_L,), 0x3F800000, jnp.int32)
    poly = [jnp.full((_L,), c, jnp.float32) for c in _LOG2_POLY]
    ln2_v = jnp.full((_L,), _LN2, jnp.float32)

    for i in range(_NCCHUNK):
        nxt = i + 1
        if nxt < _NCCHUNK:
            copies[nxt % 2] = fire(nxt, nxt % 2)
        else:
            copies[nxt % 2] = [pfire(0, nxt % 2)]
        for cp in copies[i % 2]:
            cp.wait()
        bp = bps[i % 2]
        bt = bts[i % 2]

        def row(r, carry, bp=bp, bt=bt):
            def col(cc, carry2):
                nll_a, cnt_a, pres_a = carry2
                t = bt[r, pl.ds(cc * _L, _L)]
                s = zf
                sel = zf
                for c in range(_C):
                    x = bp[c, r, pl.ds(cc * _L, _L)]
                    s = s + jnp.exp(x)
                    sel = jnp.where(t == cvecs[c], x, sel)
                bits = lax.bitcast_convert_type(s, jnp.int32)
                e = (bits >> c23) - c127
                mant = lax.bitcast_convert_type(
                    (bits & cmant) | cone_f, jnp.float32)
                p = poly[0]
                for coef in poly[1:]:
                    p = p * mant + coef
                ln_s = (p + e.astype(jnp.float32)) * ln2_v
                maskf = jnp.where(t != ign_v, onef, zf)
                nll_a = nll_a + (ln_s - sel) * maskf
                cnt_a = cnt_a + maskf
                pres_a = pres_a | (one << t)
                return (nll_a, cnt_a, pres_a)
            return lax.fori_loop(0, _W // _L, col, carry, unroll=2)

        nll, cnt, pres = lax.fori_loop(0, _CCROWS, row, (nll, cnt, pres))

    pcopies = [copies[_NCCHUNK % 2][0], None]
    for k in range(_NP2):
        nxt = k + 1
        if nxt < _NP2:
            pcopies[nxt % 2] = pfire(nxt, nxt % 2)
        pcopies[k % 2].wait()
        pb = pbs[k % 2]

        def prow(r, a, pb=pb):
            def pcol(cc, a2):
                v = pb[r, pl.ds(cc * _L, _L)]
                return a2 | (one << v)
            return lax.fori_loop(0, _W // _L, pcol, a, unroll=8)

        pres = lax.fori_loop(0, _P2ROWS, prow, pres)

    nllv[...] = nll
    cntv[...] = cnt
    presv[...] = pres
    pltpu.sync_copy(nllv, nll_hbm.at[wid])
    pltpu.sync_copy(cntv, cnt_hbm.at[wid])
    pltpu.sync_copy(presv, pres_hbm.at[wid])


def _combine_body(nll_tc_ref, cnt_tc_ref, nll_sc_ref, cnt_sc_ref, pres_ref,
                  out_ref):
    nll = nll_tc_ref[0, 0] + jnp.sum(nll_sc_ref[...])
    cnt = cnt_tc_ref[0, 0] + jnp.sum(cnt_sc_ref[...])
    w = pres_ref[...]
    nuniq = jnp.int32(0)
    for c in range(_C):
        nuniq = nuniq + jnp.max((w >> c) & 1)
    loss = nll / cnt
    out_ref[0, 0] = jnp.where(nuniq < 2, 0.0 * loss, loss)


def _combine_call(nll_tc, cnt_tc, nll_sc, cnt_sc, pres):
    return pl.pallas_call(
        _combine_body,
        in_specs=[
            pl.BlockSpec(memory_space=pltpu.SMEM),
            pl.BlockSpec(memory_space=pltpu.SMEM),
            pl.BlockSpec((_NW, _L), lambda: (0, 0)),
            pl.BlockSpec((_NW, _L), lambda: (0, 0)),
            pl.BlockSpec((_NW, _L), lambda: (0, 0)),
        ],
        out_specs=pl.BlockSpec(memory_space=pltpu.SMEM),
        out_shape=jax.ShapeDtypeStruct((1, 1), jnp.float32),
    )(nll_tc, cnt_tc, nll_sc, cnt_sc, pres)


def kernel(pred, target):
    nll_tc, cnt_tc = _tc_call(pred, target)
    nll_sc, cnt_sc, pres = _sc_main(pred, target)
    return _combine_call(nll_tc, cnt_tc, nll_sc, cnt_sc, pres)[0, 0]
